# baseline (speedup 1.0000x reference)
import functools

import jax
import jax.numpy as jnp
from jax import lax
from jax.experimental import pallas as pl
from jax.experimental.pallas import tpu as pltpu
from jax.experimental.pallas import tpu_sc as plsc

CAPACITY = 100000
D_MODEL = 512
BATCH = 4096

U = 4
CH_E = 32
CH_T = 48
TAIL = CAPACITY - BATCH
N_TCH = TAIL // CH_T

IMP_PIECE = 12000
IMP_LAST = TAIL - 7 * IMP_PIECE


def _build(nc, ns):
    nw = nc * ns
    emb_per_w = BATCH // (CH_E * nw)
    steps = -(-N_TCH // nw)
    n_iter = -(-steps // U)

    mesh = plsc.VectorSubcoreMesh(core_axis_name="c", subcore_axis_name="s")

    @functools.partial(
        pl.kernel,
        mesh=mesh,
        out_type=[
            jax.ShapeDtypeStruct((CAPACITY, D_MODEL), jnp.float32),
            jax.ShapeDtypeStruct((CAPACITY,), jnp.float32),
        ],
        scratch_types=[
            pltpu.VMEM((U, CH_T, D_MODEL), jnp.float32),
            pltpu.VMEM((IMP_PIECE, ), jnp.float32),
            pltpu.SemaphoreType.DMA((U,)),
            pltpu.SemaphoreType.DMA((U,)),
            pltpu.SemaphoreType.DMA,
        ],
    )
    def k(emb, sig, mem, imp, out_mem, out_imp, buf, ibuf, sem_in, sem_out,
          sem_imp):
        wid = lax.axis_index("s") * nc + lax.axis_index("c")

        @pl.when(wid == 0)
        def _():
            pltpu.make_async_copy(sig, ibuf.at[pl.ds(0, BATCH)],
                                  sem_imp).start()

        for p in range(8):
            sz = IMP_LAST if p == 7 else IMP_PIECE
            start = BATCH + p * IMP_PIECE

            @pl.when(wid == p + 1)
            def _(sz=sz, start=start):
                pltpu.make_async_copy(imp.at[pl.ds(start, sz)],
                                      ibuf.at[pl.ds(0, sz)], sem_imp).start()

        def e_in(j):
            off = pl.multiple_of((wid * emb_per_w + j) * CH_E, 8)
            return pltpu.make_async_copy(
                emb.at[pl.ds(off, CH_E)], buf.at[j, pl.ds(0, CH_E)],
                sem_in.at[j])

        def e_out(j):
            off = pl.multiple_of((wid * emb_per_w + j) * CH_E, 8)
            return pltpu.make_async_copy(
                buf.at[j, pl.ds(0, CH_E)], out_mem.at[pl.ds(off, CH_E)],
                sem_out.at[j])

        for j in range(emb_per_w):
            e_in(j).start()
        for j in range(emb_per_w):
            e_in(j).wait()
            e_out(j).start()
        for j in range(emb_per_w):
            e_out(j).wait()

        def t_in(c, j):
            off = pl.multiple_of(BATCH + c * CH_T, 8)
            return pltpu.make_async_copy(mem.at[pl.ds(off, CH_T)], buf.at[j],
                                         sem_in.at[j])

        def t_out(c, j):
            off = pl.multiple_of(BATCH + c * CH_T, 8)
            return pltpu.make_async_copy(buf.at[j], out_mem.at[pl.ds(off, CH_T)],
                                         sem_out.at[j])

        def body(i, carry):
            for j in range(U):
                c = wid + (i * U + j) * nw

                @pl.when(c < N_TCH)
                def _(c=c, j=j):
                    @pl.when(i > 0)
                    def _():
                        t_out(c - U * nw, j).wait()

                    t_in(c, j).start()

            for j in range(U):
                c = wid + (i * U + j) * nw

                @pl.when(c < N_TCH)
                def _(c=c, j=j):
                    t_in(c, j).wait()
                    t_out(c, j).start()

            return carry

        lax.fori_loop(0, n_iter, body, 0)

        kk_max = (N_TCH - 1 - wid) // nw
        for j in range(U):
            @pl.when(kk_max >= j)
            def _(j=j):
                t_out(wid, j).wait()

        @pl.when(wid == 0)
        def _():
            pltpu.make_async_copy(sig, ibuf.at[pl.ds(0, BATCH)],
                                  sem_imp).wait()
            pltpu.sync_copy(ibuf.at[pl.ds(0, BATCH)], out_imp.at[pl.ds(0, BATCH)])

        for p in range(8):
            sz = IMP_LAST if p == 7 else IMP_PIECE
            start = BATCH + p * IMP_PIECE

            @pl.when(wid == p + 1)
            def _(sz=sz, start=start):
                pltpu.make_async_copy(imp.at[pl.ds(start, sz)],
                                      ibuf.at[pl.ds(0, sz)], sem_imp).wait()
                pltpu.sync_copy(ibuf.at[pl.ds(0, sz)], out_imp.at[pl.ds(start, sz)])

    return k


def kernel(embeddings, loss_signal, memory, importance):
    info = plsc.get_sparse_core_info()
    k = _build(info.num_cores, info.num_subcores)
    out_mem, out_imp = k(embeddings, loss_signal, memory, importance)
    return out_mem, out_imp


# speedup vs baseline: 1.1480x; 1.1480x over previous
import functools

import jax
import jax.numpy as jnp
from jax import lax
from jax.experimental import pallas as pl
from jax.experimental.pallas import tpu as pltpu
from jax.experimental.pallas import tpu_sc as plsc

CAPACITY = 100000
D_MODEL = 512
BATCH = 4096
TAIL = CAPACITY - BATCH


BLOCK_ROWS = 4096
NB_EMB = BATCH // BLOCK_ROWS
GRID = (CAPACITY + BLOCK_ROWS - 1) // BLOCK_ROWS


def _mem_body(emb_ref, mem_ref, out_ref):
    i = pl.program_id(0)

    @pl.when(i < NB_EMB)
    def _():
        out_ref[...] = emb_ref[...]

    @pl.when(i >= NB_EMB)
    def _():
        out_ref[...] = mem_ref[...]


def _copy_memory(embeddings, memory):
    emb_last = NB_EMB - 1
    return pl.pallas_call(
        _mem_body,
        grid=(GRID,),
        in_specs=[
            pl.BlockSpec((BLOCK_ROWS, D_MODEL), lambda i: (jnp.minimum(i, emb_last), 0)),
            pl.BlockSpec((BLOCK_ROWS, D_MODEL), lambda i: (jnp.maximum(i, NB_EMB), 0)),
        ],
        out_specs=pl.BlockSpec((BLOCK_ROWS, D_MODEL), lambda i: (i, 0)),
        out_shape=jax.ShapeDtypeStruct((CAPACITY, D_MODEL), jnp.float32),
    )(embeddings, memory)


IMP_PIECE = 12000
IMP_LAST = TAIL - 7 * IMP_PIECE


def _build_imp(nc, ns):
    mesh = plsc.VectorSubcoreMesh(core_axis_name="c", subcore_axis_name="s")

    @functools.partial(
        pl.kernel,
        mesh=mesh,
        out_type=jax.ShapeDtypeStruct((CAPACITY,), jnp.float32),
        scratch_types=[pltpu.VMEM((IMP_PIECE,), jnp.float32)],
    )
    def k(sig, imp, out_imp, ibuf):
        wid = lax.axis_index("s") * nc + lax.axis_index("c")

        @pl.when(wid == 0)
        def _():
            pltpu.sync_copy(sig, ibuf.at[pl.ds(0, BATCH)])
            pltpu.sync_copy(ibuf.at[pl.ds(0, BATCH)], out_imp.at[pl.ds(0, BATCH)])

        for p in range(8):
            sz = IMP_LAST if p == 7 else IMP_PIECE
            start = BATCH + p * IMP_PIECE

            @pl.when(wid == p + 1)
            def _(sz=sz, start=start):
                pltpu.sync_copy(imp.at[pl.ds(start, sz)], ibuf.at[pl.ds(0, sz)])
                pltpu.sync_copy(ibuf.at[pl.ds(0, sz)], out_imp.at[pl.ds(start, sz)])

    return k


def kernel(embeddings, loss_signal, memory, importance):
    info = plsc.get_sparse_core_info()
    out_imp = _build_imp(info.num_cores, info.num_subcores)(
        loss_signal, importance)
    out_mem = _copy_memory(embeddings, memory)
    return out_mem, out_imp


# speedup vs baseline: 1.2894x; 1.1232x over previous
import jax
import jax.numpy as jnp
from jax.experimental import pallas as pl
from jax.experimental.pallas import tpu as pltpu

CAPACITY = 100000
D_MODEL = 512
BATCH = 4096

BLOCK_ROWS = 4096
NB_EMB = BATCH // BLOCK_ROWS
GRID = (CAPACITY + BLOCK_ROWS - 1) // BLOCK_ROWS


def _body(emb_ref, sig_ref, mem_ref, imp_ref, out_mem_ref, out_imp_ref):
    i = pl.program_id(0)

    @pl.when(i < NB_EMB)
    def _():
        out_mem_ref[...] = emb_ref[...]
        out_imp_ref[...] = sig_ref[...]

    @pl.when(i >= NB_EMB)
    def _():
        out_mem_ref[...] = mem_ref[...]
        out_imp_ref[...] = imp_ref[...]


def kernel(embeddings, loss_signal, memory, importance):
    emb_last = NB_EMB - 1
    out_mem, out_imp = pl.pallas_call(
        _body,
        grid=(GRID,),
        in_specs=[
            pl.BlockSpec((BLOCK_ROWS, D_MODEL), lambda i: (jnp.minimum(i, emb_last), 0)),
            pl.BlockSpec((BLOCK_ROWS,), lambda i: (jnp.minimum(i, emb_last),)),
            pl.BlockSpec((BLOCK_ROWS, D_MODEL), lambda i: (jnp.maximum(i, NB_EMB), 0)),
            pl.BlockSpec((BLOCK_ROWS,), lambda i: (jnp.maximum(i, NB_EMB),)),
        ],
        out_specs=[
            pl.BlockSpec((BLOCK_ROWS, D_MODEL), lambda i: (i, 0)),
            pl.BlockSpec((BLOCK_ROWS,), lambda i: (i,)),
        ],
        out_shape=[
            jax.ShapeDtypeStruct((CAPACITY, D_MODEL), jnp.float32),
            jax.ShapeDtypeStruct((CAPACITY,), jnp.float32),
        ],
        compiler_params=pltpu.CompilerParams(
            dimension_semantics=("parallel",)),
    )(embeddings, loss_signal, memory, importance)

    return out_mem, out_imp
